# baseline (speedup 1.0000x reference)
import functools

import jax
import jax.numpy as jnp
from jax import lax
from jax.experimental import pallas as pl
from jax.experimental.pallas import tpu as pltpu
from jax.experimental.pallas import tpu_sc as plsc

K = 20
N = 2048
B = 4
ROWS = 256
BF = jnp.bfloat16

_NEG = -3.0e38


def _lrelu(x):
    return jnp.where(x >= 0, x, 0.2 * x)


def _knn_body(xtb_ref, xtf_ref, idx_ref):
    b = pl.program_id(0)
    xtb = xtb_ref[0]
    xtf = xtf_ref[0]
    gram = lax.dot_general(xtb.astype(BF), xtf.astype(BF),
                           (((1,), (1,)), ((), ())),
                           preferred_element_type=jnp.float32)
    inner = -2.0 * gram
    sq = xtf * xtf
    ones = jnp.ones((1, xtf.shape[1]), jnp.float32)
    xxj = lax.dot_general(ones, sq, (((1,), (1,)), ((), ())),
                          preferred_element_type=jnp.float32,
                          precision=lax.Precision.HIGHEST)
    xxn = jnp.sum(xtb * xtb, axis=1, keepdims=True)
    score = (-xxj - inner) - xxn
    iota_col = lax.broadcasted_iota(jnp.int32, (N, 1), 0).astype(jnp.float32)
    kiota = lax.broadcasted_iota(jnp.int32, (ROWS, K), 1)
    acc = jnp.zeros((ROWS, K), jnp.int32)
    for k in range(K):
        m = jnp.max(score, axis=1, keepdims=True)
        eq = score == m
        eqf = jnp.where(eq, 1.0, 0.0)
        idxf = lax.dot_general(eqf, iota_col, (((1,), (0,)), ((), ())),
                               preferred_element_type=jnp.float32,
                               precision=lax.Precision.HIGHEST)
        idxk = jnp.minimum(idxf.astype(jnp.int32), N - 1)
        acc = jnp.where(kiota == k, idxk, acc)
        if k + 1 < K:
            score = jnp.where(eq, _NEG, score)
    idx_ref[0] = acc + b * N


def _knn(xt):
    c = xt.shape[-1]
    bb = xt.shape[0]
    return pl.pallas_call(
        _knn_body,
        grid=(bb, N // ROWS),
        in_specs=[
            pl.BlockSpec((1, ROWS, c), lambda b, i: (b, i, 0)),
            pl.BlockSpec((1, N, c), lambda b, i: (b, 0, 0)),
        ],
        out_specs=pl.BlockSpec((1, ROWS, K), lambda b, i: (b, i, 0)),
        out_shape=jax.ShapeDtypeStruct((bb, N, K), jnp.int32),
    )(xt, xt)


_SC_INFO = plsc.get_sparse_core_info()
_NC = _SC_INFO.num_cores
_NS = _SC_INFO.num_subcores
_NW = _NC * _NS
_GCH = 80
_NBUF = 4


def _gfeat_body(gpw, gnc, table_hbm, idx_hbm, out_hbm, idx_all,
                buf0, buf1, buf2, buf3,
                gs0, gs1, gs2, gs3, ss0, ss1, ss2, ss3):
    bufs = (buf0, buf1, buf2, buf3)
    gs = (gs0, gs1, gs2, gs3)
    ss = (ss0, ss1, ss2, ss3)
    wid = lax.axis_index("s") * _NC + lax.axis_index("c")
    base = wid * gpw

    pltpu.sync_copy(idx_hbm.at[pl.ds(base, gpw)], idx_all)

    def fire_g(chunk, q):
        pltpu.async_copy(
            table_hbm.at[idx_all.at[pl.ds(chunk * _GCH, _GCH)]], bufs[q], gs[q])

    def wait_g(q):
        pltpu.make_async_copy(
            table_hbm.at[idx_all.at[pl.ds(0, _GCH)]], bufs[q], gs[q]).wait()

    def fire_s(chunk, q):
        pltpu.async_copy(
            bufs[q], out_hbm.at[pl.ds(base + chunk * _GCH, _GCH)], ss[q])

    def wait_s(q):
        pltpu.make_async_copy(
            bufs[q], out_hbm.at[pl.ds(0, _GCH)], ss[q]).wait()

    for q in range(_NBUF):
        fire_g(q, q)

    def loop(g, carry):
        for q in range(_NBUF):
            c = _NBUF * g + q
            wait_g(q)
            fire_s(c, q)
        for q in range(_NBUF):
            nxt = _NBUF * (g + 1) + q

            @pl.when(nxt < gnc)
            def _(q=q, nxt=nxt):
                wait_s(q)
                fire_g(nxt, q)
        return carry

    lax.fori_loop(0, gnc // _NBUF, loop, 0)
    for q in range(_NBUF):
        wait_s(q)


def _gfeat(table, idxf):
    c = table.shape[-1]
    nrows = idxf.shape[0]
    gpw = nrows // _NW
    gnc = gpw // _GCH
    mesh = plsc.VectorSubcoreMesh(core_axis_name="c", subcore_axis_name="s")
    kfn = functools.partial(
        pl.kernel,
        mesh=mesh,
        compiler_params=pltpu.CompilerParams(use_tc_tiling_on_sc=False),
        out_type=jax.ShapeDtypeStruct((nrows, c), jnp.float32),
        scratch_types=(
            [pltpu.VMEM((gpw,), jnp.int32)]
            + [pltpu.VMEM((_GCH, c), jnp.float32)] * _NBUF
            + [pltpu.SemaphoreType.DMA] * (2 * _NBUF)
        ),
    )(functools.partial(_gfeat_body, gpw, gnc))
    return kfn(table, idxf)


def _edge_body(feat_ref, xt_ref, w_ref, m_ref, s_ref, b_ref, out_ref):
    c = xt_ref.shape[-1]
    o = w_ref.shape[-1]
    feat = feat_ref[...]
    xc = xt_ref[0]
    xcb = jnp.broadcast_to(xc[:, None, :], (ROWS, K, c))
    e1 = (feat.reshape(ROWS, K, c) - xcb).reshape(ROWS * K, c)
    ecat = jnp.concatenate([e1, xcb.reshape(ROWS * K, c)], axis=1)
    y = lax.dot_general(ecat.astype(BF), w_ref[...],
                        (((1,), (0,)), ((), ())),
                        preferred_element_type=jnp.float32)
    y = (y - m_ref[...]) * s_ref[...] + b_ref[...]
    y = _lrelu(y)
    out_ref[0] = jnp.max(y.reshape(ROWS, K, o), axis=1)


def _edge(feat, xt, wt_bf, mv, sv, bv):
    c = xt.shape[-1]
    o = wt_bf.shape[-1]
    bb = xt.shape[0]
    nb = N // ROWS
    return pl.pallas_call(
        _edge_body,
        grid=(bb, nb),
        in_specs=[
            pl.BlockSpec((ROWS * K, c), lambda b, i: (b * nb + i, 0)),
            pl.BlockSpec((1, ROWS, c), lambda b, i: (b, i, 0)),
            pl.BlockSpec((2 * c, o), lambda b, i: (0, 0)),
            pl.BlockSpec((1, o), lambda b, i: (0, 0)),
            pl.BlockSpec((1, o), lambda b, i: (0, 0)),
            pl.BlockSpec((1, o), lambda b, i: (0, 0)),
        ],
        out_specs=pl.BlockSpec((1, ROWS, o), lambda b, i: (b, i, 0)),
        out_shape=jax.ShapeDtypeStruct((bb, N, o), jnp.float32),
    )(feat, xt, wt_bf, mv, sv, bv)


def _tail_body(x1, x2, x3, x4, x5, w6, m6, s6, b6, w7, m7, s7, b7,
               w8, m8, s8, b8, w9, b9, out_ref):
    xc = jnp.concatenate(
        [x1[0], x2[0], x3[0], x4[0], x5[0]], axis=1)
    h6 = _lrelu((lax.dot_general(xc.astype(BF), w6[...],
                                 (((1,), (0,)), ((), ())),
                                 preferred_element_type=jnp.float32)
                 - m6[...]) * s6[...] + b6[...])
    g = jnp.max(h6, axis=0, keepdims=True)
    h7in = jnp.concatenate(
        [xc, jnp.broadcast_to(g, (N, g.shape[1]))], axis=1)
    h7 = _lrelu((lax.dot_general(h7in.astype(BF), w7[...],
                                 (((1,), (0,)), ((), ())),
                                 preferred_element_type=jnp.float32)
                 - m7[...]) * s7[...] + b7[...])
    h8 = _lrelu((lax.dot_general(h7.astype(BF), w8[...],
                                 (((1,), (0,)), ((), ())),
                                 preferred_element_type=jnp.float32)
                 - m8[...]) * s8[...] + b8[...])
    out = lax.dot_general(w9[...], h8.astype(BF), (((1,), (1,)), ((), ())),
                          preferred_element_type=jnp.float32) + b9[...]
    out_ref[0] = out


def _tail(xs, w6, m6, s6, b6, w7, m7, s7, b7, w8, m8, s8, b8, w9, b9):
    specs = [pl.BlockSpec((1, N, x.shape[-1]), lambda b: (b, 0, 0)) for x in xs]
    warrs = [w6, m6, s6, b6, w7, m7, s7, b7, w8, m8, s8, b8, w9, b9]
    wspecs = [pl.BlockSpec(w.shape, lambda b: (0,) * w.ndim) for w in warrs]
    return pl.pallas_call(
        _tail_body,
        grid=(B,),
        in_specs=specs + wspecs,
        out_specs=pl.BlockSpec((1, 5, N), lambda b: (b, 0, 0)),
        out_shape=jax.ShapeDtypeStruct((B, 5, N), jnp.float32),
    )(*xs, *warrs)


def _bnvecs(p, i):
    s = p['g' + i] / jnp.sqrt(p['v' + i] + 1e-5)
    return p['m' + i][None, :], s[None, :], p['b' + i][None, :]


def _edge_layer(xt, p, i):
    c = xt.shape[-1]
    w = p['W' + i]
    o = w.shape[0]
    idx = _knn(xt)
    feat = _gfeat(xt.reshape(-1, c), idx.reshape(-1))
    if c == 16:
        wt = jnp.zeros((32, o), jnp.float32)
        wt = wt.at[0:3].set(w[:, 0:3].T).at[16:19].set(w[:, 3:6].T)
    else:
        wt = w.T
    mv, sv, bv = _bnvecs(p, i)
    return _edge(feat, xt, wt.astype(BF), mv, sv, bv)


def kernel(x, params):
    p = params
    xt = jnp.swapaxes(x, 1, 2)
    xt = jnp.pad(xt, ((0, 0), (0, 0), (0, 13)))
    outs = []
    cur = xt
    for i in ['1', '2', '3', '4', '5']:
        cur = _edge_layer(cur, p, i)
        outs.append(cur)

    m6, s6, b6 = _bnvecs(p, '6')
    m7, s7, b7 = _bnvecs(p, '7')
    m8, s8, b8 = _bnvecs(p, '8')
    return _tail(outs,
                 p['W6'].T.astype(BF), m6, s6, b6,
                 p['W7'].T.astype(BF), m7, s7, b7,
                 p['W8'].T.astype(BF), m8, s8, b8,
                 p['W9'].astype(BF), p['bias9'][:, None])


# speedup vs baseline: 3.4040x; 3.4040x over previous
import functools

import jax
import jax.numpy as jnp
from jax import lax
from jax.experimental import pallas as pl
from jax.experimental.pallas import tpu as pltpu
from jax.experimental.pallas import tpu_sc as plsc

K = 20
N = 2048
B = 4
ROWS = 256
BF = jnp.bfloat16

_NEG = -3.0e38


def _lrelu(x):
    return jnp.where(x >= 0, x, 0.2 * x)


def _knn_body(xtb_ref, xtf_ref, idx_ref):
    b = pl.program_id(0)
    xtb = xtb_ref[0]
    xtf = xtf_ref[0]
    gram = lax.dot_general(xtb.astype(BF), xtf.astype(BF),
                           (((1,), (1,)), ((), ())),
                           preferred_element_type=jnp.float32)
    inner = -2.0 * gram
    sq = xtf * xtf
    ones = jnp.ones((1, xtf.shape[1]), jnp.float32)
    xxj = lax.dot_general(ones, sq, (((1,), (1,)), ((), ())),
                          preferred_element_type=jnp.float32,
                          precision=lax.Precision.HIGHEST)
    xxn = jnp.sum(xtb * xtb, axis=1, keepdims=True)
    score = (-xxj - inner) - xxn
    iota = lax.broadcasted_iota(jnp.int32, (ROWS, N), 1)
    kiota = lax.broadcasted_iota(jnp.int32, (ROWS, K), 1)
    acc = jnp.zeros((ROWS, K), jnp.int32)
    for k in range(K):
        m = jnp.max(score, axis=1, keepdims=True)
        eq = score == m
        idxk = jnp.min(jnp.where(eq, iota, N), axis=1, keepdims=True)
        acc = jnp.where(kiota == k, idxk, acc)
        if k + 1 < K:
            score = jnp.where(eq, _NEG, score)
    idx_ref[0] = acc + b * N


def _knn(xt):
    c = xt.shape[-1]
    bb = xt.shape[0]
    return pl.pallas_call(
        _knn_body,
        grid=(bb, N // ROWS),
        in_specs=[
            pl.BlockSpec((1, ROWS, c), lambda b, i: (b, i, 0)),
            pl.BlockSpec((1, N, c), lambda b, i: (b, 0, 0)),
        ],
        out_specs=pl.BlockSpec((1, ROWS, K), lambda b, i: (b, i, 0)),
        out_shape=jax.ShapeDtypeStruct((bb, N, K), jnp.int32),
    )(xt, xt)


_SC_INFO = plsc.get_sparse_core_info()
_NC = _SC_INFO.num_cores
_NS = _SC_INFO.num_subcores
_NW = _NC * _NS
_GCH = 80
_NBUF = 4


def _gfeat_body(gpw, gnc, table_hbm, idx_hbm, out_hbm, idx_all,
                buf0, buf1, buf2, buf3,
                gs0, gs1, gs2, gs3, ss0, ss1, ss2, ss3):
    bufs = (buf0, buf1, buf2, buf3)
    gs = (gs0, gs1, gs2, gs3)
    ss = (ss0, ss1, ss2, ss3)
    wid = lax.axis_index("s") * _NC + lax.axis_index("c")
    base = wid * gpw

    pltpu.sync_copy(idx_hbm.at[pl.ds(base, gpw)], idx_all)

    def fire_g(chunk, q):
        pltpu.async_copy(
            table_hbm.at[idx_all.at[pl.ds(chunk * _GCH, _GCH)]], bufs[q], gs[q])

    def wait_g(q):
        pltpu.make_async_copy(
            table_hbm.at[idx_all.at[pl.ds(0, _GCH)]], bufs[q], gs[q]).wait()

    def fire_s(chunk, q):
        pltpu.async_copy(
            bufs[q], out_hbm.at[pl.ds(base + chunk * _GCH, _GCH)], ss[q])

    def wait_s(q):
        pltpu.make_async_copy(
            bufs[q], out_hbm.at[pl.ds(0, _GCH)], ss[q]).wait()

    for q in range(_NBUF):
        fire_g(q, q)

    def loop(g, carry):
        for q in range(_NBUF):
            c = _NBUF * g + q
            wait_g(q)
            fire_s(c, q)
        for q in range(_NBUF):
            nxt = _NBUF * (g + 1) + q

            @pl.when(nxt < gnc)
            def _(q=q, nxt=nxt):
                wait_s(q)
                fire_g(nxt, q)
        return carry

    lax.fori_loop(0, gnc // _NBUF, loop, 0)
    for q in range(_NBUF):
        wait_s(q)


def _gfeat(table, idxf):
    c = table.shape[-1]
    nrows = idxf.shape[0]
    gpw = nrows // _NW
    gnc = gpw // _GCH
    mesh = plsc.VectorSubcoreMesh(core_axis_name="c", subcore_axis_name="s")
    kfn = functools.partial(
        pl.kernel,
        mesh=mesh,
        compiler_params=pltpu.CompilerParams(use_tc_tiling_on_sc=False),
        out_type=jax.ShapeDtypeStruct((nrows, c), jnp.float32),
        scratch_types=(
            [pltpu.VMEM((gpw,), jnp.int32)]
            + [pltpu.VMEM((_GCH, c), jnp.float32)] * _NBUF
            + [pltpu.SemaphoreType.DMA] * (2 * _NBUF)
        ),
    )(functools.partial(_gfeat_body, gpw, gnc))
    return kfn(table, idxf)


def _edge_body(feat_ref, xt_ref, w_ref, m_ref, s_ref, b_ref, out_ref):
    c = xt_ref.shape[-1]
    o = w_ref.shape[-1]
    feat = feat_ref[...]
    xc = xt_ref[0]
    xcb = jnp.broadcast_to(xc[:, None, :], (ROWS, K, c))
    e1 = (feat.reshape(ROWS, K, c) - xcb).reshape(ROWS * K, c)
    ecat = jnp.concatenate([e1, xcb.reshape(ROWS * K, c)], axis=1)
    y = lax.dot_general(ecat.astype(BF), w_ref[...],
                        (((1,), (0,)), ((), ())),
                        preferred_element_type=jnp.float32)
    y = (y - m_ref[...]) * s_ref[...] + b_ref[...]
    y = _lrelu(y)
    out_ref[0] = jnp.max(y.reshape(ROWS, K, o), axis=1)


def _edge(feat, xt, wt_bf, mv, sv, bv):
    c = xt.shape[-1]
    o = wt_bf.shape[-1]
    bb = xt.shape[0]
    nb = N // ROWS
    return pl.pallas_call(
        _edge_body,
        grid=(bb, nb),
        in_specs=[
            pl.BlockSpec((ROWS * K, c), lambda b, i: (b * nb + i, 0)),
            pl.BlockSpec((1, ROWS, c), lambda b, i: (b, i, 0)),
            pl.BlockSpec((2 * c, o), lambda b, i: (0, 0)),
            pl.BlockSpec((1, o), lambda b, i: (0, 0)),
            pl.BlockSpec((1, o), lambda b, i: (0, 0)),
            pl.BlockSpec((1, o), lambda b, i: (0, 0)),
        ],
        out_specs=pl.BlockSpec((1, ROWS, o), lambda b, i: (b, i, 0)),
        out_shape=jax.ShapeDtypeStruct((bb, N, o), jnp.float32),
    )(feat, xt, wt_bf, mv, sv, bv)


def _tail_body(x1, x2, x3, x4, x5, w6, m6, s6, b6, w7, m7, s7, b7,
               w8, m8, s8, b8, w9, b9, out_ref):
    xc = jnp.concatenate(
        [x1[0], x2[0], x3[0], x4[0], x5[0]], axis=1)
    h6 = _lrelu((lax.dot_general(xc.astype(BF), w6[...],
                                 (((1,), (0,)), ((), ())),
                                 preferred_element_type=jnp.float32)
                 - m6[...]) * s6[...] + b6[...])
    g = jnp.max(h6, axis=0, keepdims=True)
    h7in = jnp.concatenate(
        [xc, jnp.broadcast_to(g, (N, g.shape[1]))], axis=1)
    h7 = _lrelu((lax.dot_general(h7in.astype(BF), w7[...],
                                 (((1,), (0,)), ((), ())),
                                 preferred_element_type=jnp.float32)
                 - m7[...]) * s7[...] + b7[...])
    h8 = _lrelu((lax.dot_general(h7.astype(BF), w8[...],
                                 (((1,), (0,)), ((), ())),
                                 preferred_element_type=jnp.float32)
                 - m8[...]) * s8[...] + b8[...])
    out = lax.dot_general(w9[...], h8.astype(BF), (((1,), (1,)), ((), ())),
                          preferred_element_type=jnp.float32) + b9[...]
    out_ref[0] = out


def _tail(xs, w6, m6, s6, b6, w7, m7, s7, b7, w8, m8, s8, b8, w9, b9):
    specs = [pl.BlockSpec((1, N, x.shape[-1]), lambda b: (b, 0, 0)) for x in xs]
    warrs = [w6, m6, s6, b6, w7, m7, s7, b7, w8, m8, s8, b8, w9, b9]
    wspecs = [pl.BlockSpec(w.shape, lambda b: (0,) * w.ndim) for w in warrs]
    return pl.pallas_call(
        _tail_body,
        grid=(B,),
        in_specs=specs + wspecs,
        out_specs=pl.BlockSpec((1, 5, N), lambda b: (b, 0, 0)),
        out_shape=jax.ShapeDtypeStruct((B, 5, N), jnp.float32),
    )(*xs, *warrs)


def _bnvecs(p, i):
    s = p['g' + i] / jnp.sqrt(p['v' + i] + 1e-5)
    return p['m' + i][None, :], s[None, :], p['b' + i][None, :]


def _edge_layer(xt, p, i):
    c = xt.shape[-1]
    w = p['W' + i]
    o = w.shape[0]
    idx = _knn(xt)
    feat = _gfeat(xt.reshape(-1, c), idx.reshape(-1))
    if c == 16:
        wt = jnp.zeros((32, o), jnp.float32)
        wt = wt.at[0:3].set(w[:, 0:3].T).at[16:19].set(w[:, 3:6].T)
    else:
        wt = w.T
    mv, sv, bv = _bnvecs(p, i)
    return _edge(feat, xt, wt.astype(BF), mv, sv, bv)


def kernel(x, params):
    p = params
    xt = jnp.swapaxes(x, 1, 2)
    xt = jnp.pad(xt, ((0, 0), (0, 0), (0, 13)))
    outs = []
    cur = xt
    for i in ['1', '2', '3', '4', '5']:
        cur = _edge_layer(cur, p, i)
        outs.append(cur)

    m6, s6, b6 = _bnvecs(p, '6')
    m7, s7, b7 = _bnvecs(p, '7')
    m8, s8, b8 = _bnvecs(p, '8')
    return _tail(outs,
                 p['W6'].T.astype(BF), m6, s6, b6,
                 p['W7'].T.astype(BF), m7, s7, b7,
                 p['W8'].T.astype(BF), m8, s8, b8,
                 p['W9'].astype(BF), p['bias9'][:, None])


# speedup vs baseline: 3.4815x; 1.0228x over previous
import functools

import jax
import jax.numpy as jnp
from jax import lax
from jax.experimental import pallas as pl
from jax.experimental.pallas import tpu as pltpu
from jax.experimental.pallas import tpu_sc as plsc

K = 20
N = 2048
B = 4
ROWS = 256
BF = jnp.bfloat16

_NEG = -3.0e38


def _lrelu(x):
    return jnp.where(x >= 0, x, 0.2 * x)


KROWS = 512


def _knn_body(xtb_ref, xtf_ref, idx_ref):
    b = pl.program_id(0)
    xtb = xtb_ref[0]
    xtf = xtf_ref[0]
    gram = lax.dot_general(xtb.astype(BF), xtf.astype(BF),
                           (((1,), (1,)), ((), ())),
                           preferred_element_type=jnp.float32)
    inner = -2.0 * gram
    sq = xtf * xtf
    ones = jnp.ones((1, xtf.shape[1]), jnp.float32)
    xxj = lax.dot_general(ones, sq, (((1,), (1,)), ((), ())),
                          preferred_element_type=jnp.float32,
                          precision=lax.Precision.HIGHEST)
    xxn = jnp.sum(xtb * xtb, axis=1, keepdims=True)
    score = (-xxj - inner) - xxn
    iota = lax.broadcasted_iota(jnp.int32, (KROWS, N), 1)
    kiota = lax.broadcasted_iota(jnp.int32, (KROWS, K), 1)
    acc = jnp.zeros((KROWS, K), jnp.int32)
    for k in range(K):
        m = jnp.max(score, axis=1, keepdims=True)
        eq = score == m
        idxk = jnp.min(jnp.where(eq, iota, N), axis=1, keepdims=True)
        acc = jnp.where(kiota == k, idxk, acc)
        if k + 1 < K:
            score = jnp.where(eq, _NEG, score)
    idx_ref[0] = acc + b * N


def _knn(xt):
    c = xt.shape[-1]
    bb = xt.shape[0]
    return pl.pallas_call(
        _knn_body,
        grid=(bb, N // KROWS),
        in_specs=[
            pl.BlockSpec((1, KROWS, c), lambda b, i: (b, i, 0)),
            pl.BlockSpec((1, N, c), lambda b, i: (b, 0, 0)),
        ],
        out_specs=pl.BlockSpec((1, KROWS, K), lambda b, i: (b, i, 0)),
        out_shape=jax.ShapeDtypeStruct((bb, N, K), jnp.int32),
    )(xt, xt)


_SC_INFO = plsc.get_sparse_core_info()
_NC = _SC_INFO.num_cores
_NS = _SC_INFO.num_subcores
_NW = _NC * _NS
_GCH = 80
_NBUF = 4


def _gfeat_body(gpw, gnc, table_hbm, idx_hbm, out_hbm, idx_all,
                buf0, buf1, buf2, buf3,
                gs0, gs1, gs2, gs3, ss0, ss1, ss2, ss3):
    bufs = (buf0, buf1, buf2, buf3)
    gs = (gs0, gs1, gs2, gs3)
    ss = (ss0, ss1, ss2, ss3)
    wid = lax.axis_index("s") * _NC + lax.axis_index("c")
    base = wid * gpw

    pltpu.sync_copy(idx_hbm.at[pl.ds(base, gpw)], idx_all)

    def fire_g(chunk, q):
        pltpu.async_copy(
            table_hbm.at[idx_all.at[pl.ds(chunk * _GCH, _GCH)]], bufs[q], gs[q])

    def wait_g(q):
        pltpu.make_async_copy(
            table_hbm.at[idx_all.at[pl.ds(0, _GCH)]], bufs[q], gs[q]).wait()

    def fire_s(chunk, q):
        pltpu.async_copy(
            bufs[q], out_hbm.at[pl.ds(base + chunk * _GCH, _GCH)], ss[q])

    def wait_s(q):
        pltpu.make_async_copy(
            bufs[q], out_hbm.at[pl.ds(0, _GCH)], ss[q]).wait()

    for q in range(_NBUF):
        fire_g(q, q)

    def loop(g, carry):
        for q in range(_NBUF):
            c = _NBUF * g + q
            wait_g(q)
            fire_s(c, q)
        for q in range(_NBUF):
            nxt = _NBUF * (g + 1) + q

            @pl.when(nxt < gnc)
            def _(q=q, nxt=nxt):
                wait_s(q)
                fire_g(nxt, q)
        return carry

    lax.fori_loop(0, gnc // _NBUF, loop, 0)
    for q in range(_NBUF):
        wait_s(q)


def _gfeat(table, idxf):
    c = table.shape[-1]
    nrows = idxf.shape[0]
    gpw = nrows // _NW
    gnc = gpw // _GCH
    mesh = plsc.VectorSubcoreMesh(core_axis_name="c", subcore_axis_name="s")
    kfn = functools.partial(
        pl.kernel,
        mesh=mesh,
        compiler_params=pltpu.CompilerParams(use_tc_tiling_on_sc=False),
        out_type=jax.ShapeDtypeStruct((nrows, c), jnp.float32),
        scratch_types=(
            [pltpu.VMEM((gpw,), jnp.int32)]
            + [pltpu.VMEM((_GCH, c), jnp.float32)] * _NBUF
            + [pltpu.SemaphoreType.DMA] * (2 * _NBUF)
        ),
    )(functools.partial(_gfeat_body, gpw, gnc))
    return kfn(table, idxf)


def _edge_body(feat_ref, xt_ref, w_ref, m_ref, s_ref, b_ref, out_ref):
    c = xt_ref.shape[-1]
    o = w_ref.shape[-1]
    feat = feat_ref[...]
    xc = xt_ref[0]
    xcb = jnp.broadcast_to(xc[:, None, :], (ROWS, K, c))
    e1 = (feat.reshape(ROWS, K, c) - xcb).reshape(ROWS * K, c)
    ecat = jnp.concatenate([e1, xcb.reshape(ROWS * K, c)], axis=1)
    y = lax.dot_general(ecat.astype(BF), w_ref[...],
                        (((1,), (0,)), ((), ())),
                        preferred_element_type=jnp.float32)
    y = (y - m_ref[...]) * s_ref[...] + b_ref[...]
    y = _lrelu(y)
    out_ref[0] = jnp.max(y.reshape(ROWS, K, o), axis=1)


def _edge(feat, xt, wt_bf, mv, sv, bv):
    c = xt.shape[-1]
    o = wt_bf.shape[-1]
    bb = xt.shape[0]
    nb = N // ROWS
    return pl.pallas_call(
        _edge_body,
        grid=(bb, nb),
        in_specs=[
            pl.BlockSpec((ROWS * K, c), lambda b, i: (b * nb + i, 0)),
            pl.BlockSpec((1, ROWS, c), lambda b, i: (b, i, 0)),
            pl.BlockSpec((2 * c, o), lambda b, i: (0, 0)),
            pl.BlockSpec((1, o), lambda b, i: (0, 0)),
            pl.BlockSpec((1, o), lambda b, i: (0, 0)),
            pl.BlockSpec((1, o), lambda b, i: (0, 0)),
        ],
        out_specs=pl.BlockSpec((1, ROWS, o), lambda b, i: (b, i, 0)),
        out_shape=jax.ShapeDtypeStruct((bb, N, o), jnp.float32),
    )(feat, xt, wt_bf, mv, sv, bv)


def _tail_body(x1, x2, x3, x4, x5, w6, m6, s6, b6, w7, m7, s7, b7,
               w8, m8, s8, b8, w9, b9, out_ref):
    xc = jnp.concatenate(
        [x1[0], x2[0], x3[0], x4[0], x5[0]], axis=1)
    h6 = _lrelu((lax.dot_general(xc.astype(BF), w6[...],
                                 (((1,), (0,)), ((), ())),
                                 preferred_element_type=jnp.float32)
                 - m6[...]) * s6[...] + b6[...])
    g = jnp.max(h6, axis=0, keepdims=True)
    h7in = jnp.concatenate(
        [xc, jnp.broadcast_to(g, (N, g.shape[1]))], axis=1)
    h7 = _lrelu((lax.dot_general(h7in.astype(BF), w7[...],
                                 (((1,), (0,)), ((), ())),
                                 preferred_element_type=jnp.float32)
                 - m7[...]) * s7[...] + b7[...])
    h8 = _lrelu((lax.dot_general(h7.astype(BF), w8[...],
                                 (((1,), (0,)), ((), ())),
                                 preferred_element_type=jnp.float32)
                 - m8[...]) * s8[...] + b8[...])
    out = lax.dot_general(w9[...], h8.astype(BF), (((1,), (1,)), ((), ())),
                          preferred_element_type=jnp.float32) + b9[...]
    out_ref[0] = out


def _tail(xs, w6, m6, s6, b6, w7, m7, s7, b7, w8, m8, s8, b8, w9, b9):
    specs = [pl.BlockSpec((1, N, x.shape[-1]), lambda b: (b, 0, 0)) for x in xs]
    warrs = [w6, m6, s6, b6, w7, m7, s7, b7, w8, m8, s8, b8, w9, b9]
    wspecs = [pl.BlockSpec(w.shape, lambda b: (0,) * w.ndim) for w in warrs]
    return pl.pallas_call(
        _tail_body,
        grid=(B,),
        in_specs=specs + wspecs,
        out_specs=pl.BlockSpec((1, 5, N), lambda b: (b, 0, 0)),
        out_shape=jax.ShapeDtypeStruct((B, 5, N), jnp.float32),
    )(*xs, *warrs)


def _bnvecs(p, i):
    s = p['g' + i] / jnp.sqrt(p['v' + i] + 1e-5)
    return p['m' + i][None, :], s[None, :], p['b' + i][None, :]


def _edge_layer(xt, p, i):
    c = xt.shape[-1]
    w = p['W' + i]
    o = w.shape[0]
    idx = _knn(xt)
    feat = _gfeat(xt.reshape(-1, c), idx.reshape(-1))
    if c == 16:
        wt = jnp.zeros((32, o), jnp.float32)
        wt = wt.at[0:3].set(w[:, 0:3].T).at[16:19].set(w[:, 3:6].T)
    else:
        wt = w.T
    mv, sv, bv = _bnvecs(p, i)
    return _edge(feat, xt, wt.astype(BF), mv, sv, bv)


def kernel(x, params):
    p = params
    xt = jnp.swapaxes(x, 1, 2)
    xt = jnp.pad(xt, ((0, 0), (0, 0), (0, 13)))
    outs = []
    cur = xt
    for i in ['1', '2', '3', '4', '5']:
        cur = _edge_layer(cur, p, i)
        outs.append(cur)

    m6, s6, b6 = _bnvecs(p, '6')
    m7, s7, b7 = _bnvecs(p, '7')
    m8, s8, b8 = _bnvecs(p, '8')
    return _tail(outs,
                 p['W6'].T.astype(BF), m6, s6, b6,
                 p['W7'].T.astype(BF), m7, s7, b7,
                 p['W8'].T.astype(BF), m8, s8, b8,
                 p['W9'].astype(BF), p['bias9'][:, None])
